# baseline (speedup 1.0000x reference)
import jax
import jax.numpy as jnp
from jax.experimental import pallas as pl

N_STATES = 100000
N_SAMPLES = 1000
N_WARMUP = 100
STEPS_PER_SAMPLE = 5

LANES = 128
CHUNK_SUB = 80
N_CHUNKS = 10
SUBROWS = CHUNK_SUB * N_CHUNKS
PADDED = SUBROWS * LANES

KS0 = 64467757
KS1 = 2916123636
KS2 = (KS0 ^ KS1 ^ 0x1BD11BDA) & 0xFFFFFFFF

_ROT_A = (13, 15, 26, 6)
_ROT_B = (17, 29, 16, 24)


N_HALVES = 2
ROWS_PER_HALF = N_SAMPLES // N_HALVES


def _sampler_kernel(lw_ref, counts_ref):
    h = pl.program_id(0)
    q = pl.program_id(1)
    p = h * ROWS_PER_HALF + q

    @pl.when(q == 0)
    def _init():
        counts_ref[...] = jnp.zeros_like(counts_ref)

    t = N_WARMUP + STEPS_PER_SAMPLE * p
    base = t * N_STATES

    i = jax.lax.broadcasted_iota(jnp.int32, (CHUNK_SUB, LANES), 0)
    c = jax.lax.broadcasted_iota(jnp.int32, (CHUNK_SUB, LANES), 1)
    flat0 = i * LANES + c

    def chunk(k, carry):
        best_v, best_j = carry
        flat = flat0 + k * (CHUNK_SUB * LANES)
        ctr = (flat + base).astype(jnp.uint32)

        ks = (KS0, KS1, KS2)
        x0 = jnp.full((CHUNK_SUB, LANES), jnp.uint32(KS0), dtype=jnp.uint32)
        x1 = ctr + jnp.uint32(KS1)
        rots = (_ROT_A, _ROT_B)
        for rnd in range(5):
            for r in rots[rnd % 2]:
                x0 = x0 + x1
                x1 = jax.lax.shift_left(x1, jnp.uint32(r)) | \
                    jax.lax.shift_right_logical(x1, jnp.uint32(32 - r))
                x1 = x0 ^ x1
            x0 = x0 + jnp.uint32(ks[(rnd + 1) % 3])
            x1 = x1 + jnp.uint32((ks[(rnd + 2) % 3] + rnd + 1) & 0xFFFFFFFF)
        bits = x0 ^ x1

        fb = jax.lax.shift_right_logical(bits, jnp.uint32(9)) | \
            jnp.uint32(0x3F800000)
        u = jax.lax.bitcast_convert_type(fb, jnp.float32) - jnp.float32(1.0)
        tiny = jnp.float32(jnp.finfo(jnp.float32).tiny)
        u = jnp.maximum(tiny, u + tiny)
        g = -jnp.log(-jnp.log(u))

        score = g + lw_ref[pl.ds(k * CHUNK_SUB, CHUNK_SUB), :]
        upd = score > best_v
        return (jnp.where(upd, score, best_v), jnp.where(upd, flat, best_j))

    neg_inf = jnp.full((CHUNK_SUB, LANES), -jnp.inf, dtype=jnp.float32)
    zero_j = jnp.zeros((CHUNK_SUB, LANES), dtype=jnp.int32)
    best_v, best_j = jax.lax.fori_loop(0, N_CHUNKS, chunk, (neg_inf, zero_j))

    m = jnp.max(best_v)
    winner = jnp.min(jnp.where(best_v == m, best_j, jnp.int32(2**30)))

    i_all = jax.lax.broadcasted_iota(jnp.int32, (SUBROWS, LANES), 0)
    c_all = jax.lax.broadcasted_iota(jnp.int32, (SUBROWS, LANES), 1)
    flat_all = i_all * LANES + c_all
    counts_ref[0] += (flat_all == winner).astype(jnp.float32)


def _draw_counts(lw_pad):
    from jax.experimental.pallas import tpu as pltpu
    halves = pl.pallas_call(
        _sampler_kernel,
        grid=(N_HALVES, ROWS_PER_HALF),
        in_specs=[pl.BlockSpec((SUBROWS, LANES), lambda h, q: (0, 0))],
        out_specs=pl.BlockSpec((1, SUBROWS, LANES), lambda h, q: (h, 0, 0)),
        out_shape=jax.ShapeDtypeStruct((N_HALVES, SUBROWS, LANES), jnp.float32),
        compiler_params=pltpu.CompilerParams(
            dimension_semantics=("parallel", "arbitrary")),
    )(lw_pad)
    return halves[0] + halves[1]


def kernel(A, D, observation):
    likelihood = A[observation, :]
    posterior_weights = likelihood * D
    posterior_weights = posterior_weights / (jnp.sum(posterior_weights) + 1e-16)
    log_weights = jnp.log(posterior_weights + 1e-16)
    lw_pad = jnp.concatenate(
        [log_weights,
         jnp.full((PADDED - N_STATES,), -jnp.inf, dtype=jnp.float32)]
    ).reshape(SUBROWS, LANES)

    counts = _draw_counts(lw_pad)

    counts_flat = counts.reshape(-1)[:N_STATES]
    posterior_estimate = counts_flat / float(N_SAMPLES)
    return posterior_estimate / (jnp.sum(posterior_estimate) + 1e-16)


# speedup vs baseline: 1.0176x; 1.0176x over previous
import functools

import jax
import jax.numpy as jnp
from jax.experimental import pallas as pl
from jax.experimental.pallas import tpu as pltpu
from jax.experimental.pallas import tpu_sc as plsc

N_STATES = 100000
N_SAMPLES = 1000
N_WARMUP = 100
STEPS_PER_SAMPLE = 5

LANES = 128
CHUNK_SUB = 80
N_CHUNKS = 10
SUBROWS = CHUNK_SUB * N_CHUNKS
PADDED = SUBROWS * LANES

KS0 = 64467757
KS1 = 2916123636
KS2 = (KS0 ^ KS1 ^ 0x1BD11BDA) & 0xFFFFFFFF

_ROT_A = (13, 15, 26, 6)
_ROT_B = (17, 29, 16, 24)


SC_LANES = 16
PAD_SAMPLES = 1008
HIST_PAD = N_STATES + SC_LANES


def _sampler_kernel(lw_ref, idx_ref):
    p = pl.program_id(0)

    @pl.when(p == 0)
    def _init():
        idx_ref[...] = jnp.full_like(idx_ref, N_STATES)

    t = N_WARMUP + STEPS_PER_SAMPLE * p
    base = t * N_STATES

    i = jax.lax.broadcasted_iota(jnp.int32, (CHUNK_SUB, LANES), 0)
    c = jax.lax.broadcasted_iota(jnp.int32, (CHUNK_SUB, LANES), 1)
    flat0 = i * LANES + c

    def chunk(k, carry):
        best_v, best_j = carry
        flat = flat0 + k * (CHUNK_SUB * LANES)
        ctr = (flat + base).astype(jnp.uint32)

        ks = (KS0, KS1, KS2)
        x0 = jnp.full((CHUNK_SUB, LANES), jnp.uint32(KS0), dtype=jnp.uint32)
        x1 = ctr + jnp.uint32(KS1)
        rots = (_ROT_A, _ROT_B)
        for rnd in range(5):
            for r in rots[rnd % 2]:
                x0 = x0 + x1
                x1 = jax.lax.shift_left(x1, jnp.uint32(r)) | \
                    jax.lax.shift_right_logical(x1, jnp.uint32(32 - r))
                x1 = x0 ^ x1
            x0 = x0 + jnp.uint32(ks[(rnd + 1) % 3])
            x1 = x1 + jnp.uint32((ks[(rnd + 2) % 3] + rnd + 1) & 0xFFFFFFFF)
        bits = x0 ^ x1

        fb = jax.lax.shift_right_logical(bits, jnp.uint32(9)) | \
            jnp.uint32(0x3F800000)
        u = jax.lax.bitcast_convert_type(fb, jnp.float32) - jnp.float32(1.0)
        tiny = jnp.float32(jnp.finfo(jnp.float32).tiny)
        u = jnp.maximum(tiny, u + tiny)
        g = -jnp.log(-jnp.log(u))

        score = g + lw_ref[pl.ds(k * CHUNK_SUB, CHUNK_SUB), :]
        upd = score > best_v
        return (jnp.where(upd, score, best_v), jnp.where(upd, flat, best_j))

    neg_inf = jnp.full((CHUNK_SUB, LANES), -jnp.inf, dtype=jnp.float32)
    zero_j = jnp.zeros((CHUNK_SUB, LANES), dtype=jnp.int32)
    best_v, best_j = jax.lax.fori_loop(0, N_CHUNKS, chunk, (neg_inf, zero_j))

    m = jnp.max(best_v)
    winner = jnp.min(jnp.where(best_v == m, best_j, jnp.int32(2**30)))
    idx_ref[pl.ds(p, 1), :] = jnp.full((1, 1), winner, dtype=jnp.int32)


def _draw_indices(lw_pad):
    return pl.pallas_call(
        _sampler_kernel,
        grid=(N_SAMPLES,),
        in_specs=[pl.BlockSpec((SUBROWS, LANES), lambda p: (0, 0))],
        out_specs=pl.BlockSpec((PAD_SAMPLES, 1), lambda p: (0, 0)),
        out_shape=jax.ShapeDtypeStruct((PAD_SAMPLES, 1), jnp.int32),
    )(lw_pad)


@functools.partial(
    pl.kernel,
    out_type=jax.ShapeDtypeStruct((N_STATES,), jnp.float32),
    mesh=plsc.VectorSubcoreMesh(core_axis_name="c", subcore_axis_name="s"),
    compiler_params=pltpu.CompilerParams(needs_layout_passes=False),
    scratch_types=[
        pltpu.VMEM((HIST_PAD,), jnp.float32),
        pltpu.VMEM((PAD_SAMPLES,), jnp.int32),
    ],
)
def _sc_histogram(idx_hbm, zeros_hbm, out_hbm, counts_v, idx_v):
    wid = jax.lax.axis_index("s") * 2 + jax.lax.axis_index("c")

    @pl.when(wid == 0)
    def _():
        pltpu.sync_copy(zeros_hbm, counts_v)
        pltpu.sync_copy(idx_hbm, idx_v)
        ones = jnp.full((SC_LANES,), 1.0, dtype=jnp.float32)
        for i in range(PAD_SAMPLES // SC_LANES):
            v = idx_v[pl.ds(i * SC_LANES, SC_LANES)]
            plsc.addupdate_scatter(counts_v, [v], ones)
        pltpu.sync_copy(counts_v.at[pl.ds(0, N_STATES)], out_hbm)


def kernel(A, D, observation):
    likelihood = A[observation, :]
    posterior_weights = likelihood * D
    posterior_weights = posterior_weights / (jnp.sum(posterior_weights) + 1e-16)
    log_weights = jnp.log(posterior_weights + 1e-16)
    lw_pad = jnp.concatenate(
        [log_weights,
         jnp.full((PADDED - N_STATES,), -jnp.inf, dtype=jnp.float32)]
    ).reshape(SUBROWS, LANES)

    idx = _draw_indices(lw_pad).reshape(PAD_SAMPLES)
    counts = _sc_histogram(idx, jnp.zeros((HIST_PAD,), jnp.float32))

    posterior_estimate = counts / float(N_SAMPLES)
    return posterior_estimate / (jnp.sum(posterior_estimate) + 1e-16)


# speedup vs baseline: 1.1383x; 1.1186x over previous
import functools

import jax
import jax.numpy as jnp
from jax.experimental import pallas as pl
from jax.experimental.pallas import tpu as pltpu
from jax.experimental.pallas import tpu_sc as plsc

N_STATES = 100000
N_SAMPLES = 1000
N_WARMUP = 100
STEPS_PER_SAMPLE = 5

LANES = 128
CHUNK_SUB = 80
N_CHUNKS = 10
SUBROWS = CHUNK_SUB * N_CHUNKS
PADDED = SUBROWS * LANES

KS0 = 64467757
KS1 = 2916123636
KS2 = (KS0 ^ KS1 ^ 0x1BD11BDA) & 0xFFFFFFFF

_ROT_A = (13, 15, 26, 6)
_ROT_B = (17, 29, 16, 24)


SC_LANES = 16
PAD_SAMPLES = 1008
HIST_PAD = N_STATES + SC_LANES


VREGS_PER_CHUNK = CHUNK_SUB // 8


def _sampler_kernel(lw_ref, pv_ref, pj_ref):
    p = pl.program_id(0)

    @pl.when(p == 0)
    def _init():
        pv_ref[...] = jnp.full_like(pv_ref, -jnp.inf)
        pj_ref[...] = jnp.full_like(pj_ref, N_STATES)

    t = N_WARMUP + STEPS_PER_SAMPLE * p
    base = t * N_STATES

    i = jax.lax.broadcasted_iota(jnp.int32, (CHUNK_SUB, LANES), 0)
    c = jax.lax.broadcasted_iota(jnp.int32, (CHUNK_SUB, LANES), 1)
    flat0 = i * LANES + c

    def chunk(k, carry):
        best_v, best_j = carry
        flat = flat0 + k * (CHUNK_SUB * LANES)
        ctr = (flat + base).astype(jnp.uint32)

        ks = (KS0, KS1, KS2)
        x0 = jnp.full((CHUNK_SUB, LANES), jnp.uint32(KS0), dtype=jnp.uint32)
        x1 = ctr + jnp.uint32(KS1)
        rots = (_ROT_A, _ROT_B)
        for rnd in range(5):
            for r in rots[rnd % 2]:
                x0 = x0 + x1
                x1 = jax.lax.shift_left(x1, jnp.uint32(r)) | \
                    jax.lax.shift_right_logical(x1, jnp.uint32(32 - r))
                x1 = x0 ^ x1
            x0 = x0 + jnp.uint32(ks[(rnd + 1) % 3])
            x1 = x1 + jnp.uint32((ks[(rnd + 2) % 3] + rnd + 1) & 0xFFFFFFFF)
        bits = x0 ^ x1

        fb = jax.lax.shift_right_logical(bits, jnp.uint32(9)) | \
            jnp.uint32(0x3F800000)
        u = jax.lax.bitcast_convert_type(fb, jnp.float32) - jnp.float32(1.0)
        tiny = jnp.float32(jnp.finfo(jnp.float32).tiny)
        u = jnp.maximum(tiny, u + tiny)
        g = -jnp.log(-jnp.log(u))

        score = g + lw_ref[pl.ds(k * CHUNK_SUB, CHUNK_SUB), :]
        s3 = score.reshape(VREGS_PER_CHUNK, 8, LANES)
        f3 = flat.reshape(VREGS_PER_CHUNK, 8, LANES)
        for v in range(VREGS_PER_CHUNK):
            upd = s3[v] > best_v
            best_v = jnp.where(upd, s3[v], best_v)
            best_j = jnp.where(upd, f3[v], best_j)
        return (best_v, best_j)

    neg_inf = jnp.full((8, LANES), -jnp.inf, dtype=jnp.float32)
    zero_j = jnp.zeros((8, LANES), dtype=jnp.int32)
    best_v, best_j = jax.lax.fori_loop(0, N_CHUNKS, chunk, (neg_inf, zero_j))

    bv_max = jnp.max(best_v, axis=0, keepdims=True)
    eq = best_v == bv_max
    bj_min = jnp.min(jnp.where(eq, best_j, jnp.int32(2**30)), axis=0,
                     keepdims=True)
    pv_ref[pl.ds(p, 1), :] = bv_max
    pj_ref[pl.ds(p, 1), :] = bj_min


def _draw_partials(lw_pad):
    return pl.pallas_call(
        _sampler_kernel,
        grid=(N_SAMPLES,),
        in_specs=[pl.BlockSpec((SUBROWS, LANES), lambda p: (0, 0))],
        out_specs=[
            pl.BlockSpec((PAD_SAMPLES, LANES), lambda p: (0, 0)),
            pl.BlockSpec((PAD_SAMPLES, LANES), lambda p: (0, 0)),
        ],
        out_shape=[
            jax.ShapeDtypeStruct((PAD_SAMPLES, LANES), jnp.float32),
            jax.ShapeDtypeStruct((PAD_SAMPLES, LANES), jnp.int32),
        ],
    )(lw_pad)


def _lane_argmax_kernel(pv_ref, pj_ref, idx_ref):
    v = pv_ref[...]
    j = pj_ref[...]
    m = jnp.max(v, axis=1, keepdims=True)
    wj = jnp.min(jnp.where(v == m, j, jnp.int32(2**30)), axis=1, keepdims=True)
    idx_ref[...] = wj


def _lane_argmax(pv, pj):
    rows = 8
    return pl.pallas_call(
        _lane_argmax_kernel,
        grid=(PAD_SAMPLES // rows,),
        in_specs=[
            pl.BlockSpec((rows, LANES), lambda p: (p, 0)),
            pl.BlockSpec((rows, LANES), lambda p: (p, 0)),
        ],
        out_specs=pl.BlockSpec((rows, 1), lambda p: (p, 0)),
        out_shape=jax.ShapeDtypeStruct((PAD_SAMPLES, 1), jnp.int32),
    )(pv, pj)


@functools.cache
def _sc_histogram_fn():
    @functools.partial(
        pl.kernel,
        out_type=jax.ShapeDtypeStruct((N_STATES,), jnp.float32),
        mesh=plsc.VectorSubcoreMesh(core_axis_name="c", subcore_axis_name="s"),
        compiler_params=pltpu.CompilerParams(needs_layout_passes=False),
        scratch_types=[
            pltpu.VMEM((HIST_PAD,), jnp.float32),
            pltpu.VMEM((PAD_SAMPLES,), jnp.int32),
        ],
    )
    def _sc_histogram(idx_hbm, zeros_hbm, out_hbm, counts_v, idx_v):
        wid = jax.lax.axis_index("s") * 2 + jax.lax.axis_index("c")

        @pl.when(wid == 0)
        def _():
            pltpu.sync_copy(zeros_hbm, counts_v)
            pltpu.sync_copy(idx_hbm, idx_v)
            ones = jnp.full((SC_LANES,), 1.0, dtype=jnp.float32)
            for i in range(PAD_SAMPLES // SC_LANES):
                v = idx_v[pl.ds(i * SC_LANES, SC_LANES)]
                plsc.addupdate_scatter(counts_v, [v], ones)
            pltpu.sync_copy(counts_v.at[pl.ds(0, N_STATES)], out_hbm)

    return _sc_histogram


def kernel(A, D, observation):
    likelihood = A[observation, :]
    posterior_weights = likelihood * D
    posterior_weights = posterior_weights / (jnp.sum(posterior_weights) + 1e-16)
    log_weights = jnp.log(posterior_weights + 1e-16)
    lw_pad = jnp.concatenate(
        [log_weights,
         jnp.full((PADDED - N_STATES,), -jnp.inf, dtype=jnp.float32)]
    ).reshape(SUBROWS, LANES)

    pv, pj = _draw_partials(lw_pad)
    idx = _lane_argmax(pv, pj).reshape(PAD_SAMPLES)
    counts = _sc_histogram_fn()(idx, jnp.zeros((HIST_PAD,), jnp.float32))

    posterior_estimate = counts / float(N_SAMPLES)
    return posterior_estimate / (jnp.sum(posterior_estimate) + 1e-16)
